# baseline (speedup 1.0000x reference)
import functools

import jax
import jax.numpy as jnp
from jax import lax
from jax.experimental import pallas as pl
from jax.experimental.pallas import tpu as pltpu
from jax.experimental.pallas import tpu_sc as plsc

VOCAB = 27
EMB = 5
OUT = 27
PAD = 32
BATCH = 16384

NC = 2
NS = 16
NW = NC * NS
B_PER_W = BATCH // NW
CHUNK = 128
NCHUNK = B_PER_W // CHUNK
LANES = 16


def _table_body(emb_ref, w_ref, out_ref):
    emb = emb_ref[...]
    w = w_ref[...]
    t1 = jnp.dot(emb, w[0:EMB, :], preferred_element_type=jnp.float32,
                 precision=lax.Precision.HIGHEST)
    t2 = jnp.dot(emb, w[EMB:, :], preferred_element_type=jnp.float32,
                 precision=lax.Precision.HIGHEST)
    for a in range(VOCAB):
        out_ref[pl.ds(a * VOCAB, VOCAB), pl.ds(0, OUT)] = t1[a:a + 1, :] + t2


_build_table = pl.pallas_call(
    _table_body,
    out_shape=jax.ShapeDtypeStruct((VOCAB * VOCAB, PAD), jnp.float32),
)


def _sc_body(table_hbm, xs_hbm, out_hbm, xs_v, idx_v, rows_v, sem):
    wid = lax.axis_index("s") * NC + lax.axis_index("c")
    base = wid * B_PER_W
    pltpu.sync_copy(xs_hbm.at[pl.ds(base, B_PER_W), :], xs_v)
    i16 = lax.iota(jnp.int32, LANES)
    zeros = jnp.zeros((LANES,), jnp.int32)
    ones = zeros + 1
    copies = []
    for c in range(NCHUNK):
        for j in range(CHUNK // LANES):
            rows = i16 + (c * CHUNK + j * LANES)
            x0 = plsc.load_gather(xs_v, [rows, zeros])
            x1 = plsc.load_gather(xs_v, [rows, ones])
            idx_v[c, pl.ds(j * LANES, LANES)] = x0 * VOCAB + x1
        copies.append(
            pltpu.async_copy(
                table_hbm.at[idx_v.at[c]],
                rows_v.at[pl.ds(c * CHUNK, CHUNK)],
                sem,
            )
        )
    for cp in copies:
        cp.wait()
    pltpu.sync_copy(rows_v, out_hbm.at[pl.ds(base, B_PER_W)])


@functools.lru_cache(maxsize=None)
def _make_gather():
    return pl.kernel(
        _sc_body,
        out_type=jax.ShapeDtypeStruct((BATCH, PAD), jnp.float32),
        mesh=plsc.VectorSubcoreMesh(core_axis_name="c", subcore_axis_name="s"),
        compiler_params=pltpu.CompilerParams(
            needs_layout_passes=False, use_tc_tiling_on_sc=False
        ),
        scratch_types=[
            pltpu.VMEM((B_PER_W, 2), jnp.int32),
            pltpu.VMEM((NCHUNK, CHUNK), jnp.int32),
            pltpu.VMEM((B_PER_W, PAD), jnp.float32),
            pltpu.SemaphoreType.DMA,
        ],
    )


def kernel(xs, embedding, W):
    table = _build_table(embedding, W)
    return _make_gather()(table, xs)[:, :OUT]


# speedup vs baseline: 1.1682x; 1.1682x over previous
import functools

import jax
import jax.numpy as jnp
from jax import lax
from jax.experimental import pallas as pl
from jax.experimental.pallas import tpu as pltpu
from jax.experimental.pallas import tpu_sc as plsc

VOCAB = 27
EMB = 5
OUT = 27
PAD = 32
BATCH = 16384

NC = 2
NS = 16
NW = NC * NS
B_PER_W = BATCH // NW
CHUNK = 128
NCHUNK = B_PER_W // CHUNK


def _prep_body(emb_ref, w_ref, xs_ref, table_ref, idx_ref):
    emb = emb_ref[...]
    w = w_ref[...]
    t1 = jnp.dot(emb, w[0:EMB, :], preferred_element_type=jnp.float32,
                 precision=lax.Precision.HIGHEST)
    t2 = jnp.dot(emb, w[EMB:, :], preferred_element_type=jnp.float32,
                 precision=lax.Precision.HIGHEST)
    for a in range(VOCAB):
        table_ref[pl.ds(a * VOCAB, VOCAB), pl.ds(0, OUT)] = t1[a:a + 1, :] + t2
    xst = jnp.transpose(xs_ref[...])
    idx_ref[...] = xst[0, :] * VOCAB + xst[1, :]


_prep = pl.pallas_call(
    _prep_body,
    out_shape=(
        jax.ShapeDtypeStruct((VOCAB * VOCAB, PAD), jnp.float32),
        jax.ShapeDtypeStruct((BATCH,), jnp.int32),
    ),
)


def _sc_body(table_hbm, idx_hbm, out_hbm, idx_v, rows_v, sem):
    wid = lax.axis_index("s") * NC + lax.axis_index("c")
    base = wid * B_PER_W
    pltpu.sync_copy(idx_hbm.at[pl.ds(base, B_PER_W)], idx_v)
    copies = []
    for c in range(NCHUNK):
        copies.append(
            pltpu.async_copy(
                table_hbm.at[idx_v.at[pl.ds(c * CHUNK, CHUNK)]],
                rows_v.at[pl.ds(c * CHUNK, CHUNK)],
                sem,
            )
        )
    for cp in copies:
        cp.wait()
    pltpu.sync_copy(rows_v, out_hbm.at[pl.ds(base, B_PER_W)])


@functools.lru_cache(maxsize=None)
def _make_gather():
    return pl.kernel(
        _sc_body,
        out_type=jax.ShapeDtypeStruct((BATCH, PAD), jnp.float32),
        mesh=plsc.VectorSubcoreMesh(core_axis_name="c", subcore_axis_name="s"),
        compiler_params=pltpu.CompilerParams(
            needs_layout_passes=False, use_tc_tiling_on_sc=False
        ),
        scratch_types=[
            pltpu.VMEM((B_PER_W,), jnp.int32),
            pltpu.VMEM((B_PER_W, PAD), jnp.float32),
            pltpu.SemaphoreType.DMA,
        ],
    )


def kernel(xs, embedding, W):
    table, idx = _prep(embedding, W, xs)
    return _make_gather()(table, idx)[:, :OUT]


# speedup vs baseline: 1.3903x; 1.1901x over previous
import functools

import jax
import jax.numpy as jnp
from jax import lax
from jax.experimental import pallas as pl
from jax.experimental.pallas import tpu as pltpu
from jax.experimental.pallas import tpu_sc as plsc

VOCAB = 27
EMB = 5
OUT = 27
PAD = 32
BATCH = 16384

NC = 2
NS = 16
NW = NC * NS
B_PER_W = BATCH // NW
CHUNK = 128
NCHUNK = B_PER_W // CHUNK
LANES = 16


def _table_body(emb_ref, w_ref, out_ref):
    emb = emb_ref[...]
    w = w_ref[...]
    t1 = jnp.dot(emb, w[0:EMB, :], preferred_element_type=jnp.float32,
                 precision=lax.Precision.HIGHEST)
    t2 = jnp.dot(emb, w[EMB:, :], preferred_element_type=jnp.float32,
                 precision=lax.Precision.HIGHEST)
    for a in range(VOCAB):
        out_ref[pl.ds(a * VOCAB, VOCAB), pl.ds(0, OUT)] = t1[a:a + 1, :] + t2


_build_table = pl.pallas_call(
    _table_body,
    out_shape=jax.ShapeDtypeStruct((VOCAB * VOCAB, PAD), jnp.float32),
)


def _sc_body(table_hbm, xst_hbm, out_hbm, x0_v, x1_v, idx_v, rows_v, sem):
    wid = lax.axis_index("s") * NC + lax.axis_index("c")
    base = wid * B_PER_W
    pltpu.sync_copy(xst_hbm.at[0, pl.ds(base, B_PER_W)], x0_v)
    pltpu.sync_copy(xst_hbm.at[1, pl.ds(base, B_PER_W)], x1_v)
    copies = []
    for c in range(NCHUNK):
        for j in range(CHUNK // LANES):
            o = c * CHUNK + j * LANES
            idx_v[pl.ds(o, LANES)] = (
                x0_v[pl.ds(o, LANES)] * VOCAB + x1_v[pl.ds(o, LANES)]
            )
        copies.append(
            pltpu.async_copy(
                table_hbm.at[idx_v.at[pl.ds(c * CHUNK, CHUNK)]],
                rows_v.at[pl.ds(c * CHUNK, CHUNK)],
                sem,
            )
        )
    for cp in copies:
        cp.wait()
    pltpu.sync_copy(rows_v, out_hbm.at[pl.ds(base, B_PER_W)])


@functools.lru_cache(maxsize=None)
def _make_gather():
    return pl.kernel(
        _sc_body,
        out_type=jax.ShapeDtypeStruct((BATCH, PAD), jnp.float32),
        mesh=plsc.VectorSubcoreMesh(core_axis_name="c", subcore_axis_name="s"),
        compiler_params=pltpu.CompilerParams(
            needs_layout_passes=False, use_tc_tiling_on_sc=False
        ),
        scratch_types=[
            pltpu.VMEM((B_PER_W,), jnp.int32),
            pltpu.VMEM((B_PER_W,), jnp.int32),
            pltpu.VMEM((B_PER_W,), jnp.int32),
            pltpu.VMEM((B_PER_W, PAD), jnp.float32),
            pltpu.SemaphoreType.DMA,
        ],
    )


def kernel(xs, embedding, W):
    table = _build_table(embedding, W)
    return _make_gather()(table, xs.T)[:, :OUT]


# speedup vs baseline: 1.4473x; 1.0409x over previous
import functools

import jax
import jax.numpy as jnp
from jax import lax
from jax.experimental import pallas as pl
from jax.experimental.pallas import tpu as pltpu
from jax.experimental.pallas import tpu_sc as plsc

VOCAB = 27
EMB = 5
OUT = 27
PAD = 32
BATCH = 16384

NC = 2
NS = 16
NW = NC * NS
B_PER_W = BATCH // NW
CHUNK = 128
NCHUNK = B_PER_W // CHUNK
LANES = 16


def _table_body(emb_ref, w_ref, out_ref):
    emb = emb_ref[...]
    w = w_ref[...]
    t1 = jnp.dot(emb, w[0:EMB, :], preferred_element_type=jnp.float32,
                 precision=lax.Precision.HIGHEST)
    t2 = jnp.dot(emb, w[EMB:, :], preferred_element_type=jnp.float32,
                 precision=lax.Precision.HIGHEST)
    for a in range(VOCAB):
        out_ref[pl.ds(a * VOCAB, VOCAB), pl.ds(0, OUT)] = t1[a:a + 1, :] + t2


_build_table = pl.pallas_call(
    _table_body,
    out_shape=jax.ShapeDtypeStruct((VOCAB * VOCAB, PAD), jnp.float32),
)


def _sc_body(table_hbm, xst_hbm, out_hbm, x0_v, x1_v, idx_v, rows_v, tab_s, sem):
    sid = lax.axis_index("s")
    wid = sid * NC + lax.axis_index("c")
    base = wid * B_PER_W
    @pl.when(sid == 0)
    def _stage():
        pltpu.sync_copy(table_hbm, tab_s)

    pltpu.sync_copy(xst_hbm.at[0, pl.ds(base, B_PER_W)], x0_v)
    pltpu.sync_copy(xst_hbm.at[1, pl.ds(base, B_PER_W)], x1_v)
    plsc.subcore_barrier()
    copies = []
    for c in range(NCHUNK):
        for j in range(CHUNK // LANES):
            o = c * CHUNK + j * LANES
            idx_v[pl.ds(o, LANES)] = (
                x0_v[pl.ds(o, LANES)] * VOCAB + x1_v[pl.ds(o, LANES)]
            )
        copies.append(
            pltpu.async_copy(
                tab_s.at[idx_v.at[pl.ds(c * CHUNK, CHUNK)]],
                rows_v.at[pl.ds(c * CHUNK, CHUNK)],
                sem,
            )
        )
    for cp in copies:
        cp.wait()
    pltpu.sync_copy(rows_v, out_hbm.at[pl.ds(base, B_PER_W)])


@functools.lru_cache(maxsize=None)
def _make_gather():
    return pl.kernel(
        _sc_body,
        out_type=jax.ShapeDtypeStruct((BATCH, PAD), jnp.float32),
        mesh=plsc.VectorSubcoreMesh(core_axis_name="c", subcore_axis_name="s"),
        compiler_params=pltpu.CompilerParams(
            needs_layout_passes=False, use_tc_tiling_on_sc=False
        ),
        scratch_types=[
            pltpu.VMEM((B_PER_W,), jnp.int32),
            pltpu.VMEM((B_PER_W,), jnp.int32),
            pltpu.VMEM((B_PER_W,), jnp.int32),
            pltpu.VMEM((B_PER_W, PAD), jnp.float32),
            pltpu.VMEM_SHARED((VOCAB * VOCAB, PAD), jnp.float32),
            pltpu.SemaphoreType.DMA,
        ],
    )


def kernel(xs, embedding, W):
    table = _build_table(embedding, W)
    return _make_gather()(table, xs.T)[:, :OUT]


# speedup vs baseline: 1.4564x; 1.0063x over previous
import functools

import jax
import jax.numpy as jnp
from jax import lax
from jax.experimental import pallas as pl
from jax.experimental.pallas import tpu as pltpu
from jax.experimental.pallas import tpu_sc as plsc

VOCAB = 27
EMB = 5
OUT = 27
PAD = 32
ROWS = 736
BATCH = 16384

NC = 2
NS = 16
NW = NC * NS
B_PER_W = BATCH // NW
CHUNK = 128
NCHUNK = B_PER_W // CHUNK
LANES = 16
STAGE = ROWS // NS


def _table_body(emb_ref, w_ref, out_ref):
    emb = emb_ref[...]
    w = w_ref[...]
    t1 = jnp.dot(emb, w[0:EMB, :], preferred_element_type=jnp.float32,
                 precision=lax.Precision.HIGHEST)
    t2 = jnp.dot(emb, w[EMB:, :], preferred_element_type=jnp.float32,
                 precision=lax.Precision.HIGHEST)
    for a in range(VOCAB):
        out_ref[pl.ds(a * VOCAB, VOCAB), pl.ds(0, OUT)] = t1[a:a + 1, :] + t2


_build_table = pl.pallas_call(
    _table_body,
    out_shape=jax.ShapeDtypeStruct((ROWS, PAD), jnp.float32),
)


def _sc_body(table_hbm, xst_hbm, out_hbm, x0_v, x1_v, idx_v, rows_v, tab_s,
             gsem, wsem):
    sid = lax.axis_index("s")
    wid = sid * NC + lax.axis_index("c")
    base = wid * B_PER_W
    pltpu.sync_copy(
        table_hbm.at[pl.ds(sid * STAGE, STAGE), :],
        tab_s.at[pl.ds(sid * STAGE, STAGE), :],
    )
    pltpu.sync_copy(xst_hbm.at[0, pl.ds(base, B_PER_W)], x0_v)
    pltpu.sync_copy(xst_hbm.at[1, pl.ds(base, B_PER_W)], x1_v)
    for j in range(B_PER_W // LANES):
        o = j * LANES
        idx_v[pl.ds(o, LANES)] = (
            x0_v[pl.ds(o, LANES)] * VOCAB + x1_v[pl.ds(o, LANES)]
        )
    plsc.subcore_barrier()
    gathers = []
    for c in range(NCHUNK):
        gathers.append(
            pltpu.async_copy(
                tab_s.at[idx_v.at[pl.ds(c * CHUNK, CHUNK)]],
                rows_v.at[pl.ds(c * CHUNK, CHUNK)],
                gsem,
            )
        )
    writes = []
    for c in range(NCHUNK):
        gathers[c].wait()
        writes.append(
            pltpu.async_copy(
                rows_v.at[pl.ds(c * CHUNK, CHUNK)],
                out_hbm.at[pl.ds(base + c * CHUNK, CHUNK)],
                wsem,
            )
        )
    for w in writes:
        w.wait()


@functools.lru_cache(maxsize=None)
def _make_gather():
    return pl.kernel(
        _sc_body,
        out_type=jax.ShapeDtypeStruct((BATCH, PAD), jnp.float32),
        mesh=plsc.VectorSubcoreMesh(core_axis_name="c", subcore_axis_name="s"),
        compiler_params=pltpu.CompilerParams(
            needs_layout_passes=False, use_tc_tiling_on_sc=False
        ),
        scratch_types=[
            pltpu.VMEM((B_PER_W,), jnp.int32),
            pltpu.VMEM((B_PER_W,), jnp.int32),
            pltpu.VMEM((B_PER_W,), jnp.int32),
            pltpu.VMEM((B_PER_W, PAD), jnp.float32),
            pltpu.VMEM_SHARED((ROWS, PAD), jnp.float32),
            pltpu.SemaphoreType.DMA,
            pltpu.SemaphoreType.DMA,
        ],
    )


def kernel(xs, embedding, W):
    table = _build_table(embedding, W)
    return _make_gather()(table, xs.T)[:, :OUT]
